# baseline (speedup 1.0000x reference)
import jax
import jax.numpy as jnp
from jax.experimental import pallas as pl
from jax.experimental.pallas import tpu as pltpu

_OFF_BOX = 24
_OFF_OBJ = 72
_NPAD = 80
_CHUNK_SIZES = (1280, 5120, 6784, 6816)


def _chunks(n_rows):
    out, s = [], 0
    for ln in _CHUNK_SIZES:
        ln = min(ln, n_rows - s)
        if ln <= 0:
            break
        out.append((s, ln))
        s += ln
    if s < n_rows:
        out.append((s, n_rows - s))
    return out


def _heads_kernel(x_hbm, wc_ref, wb_ref, wo_ref, b_ref,
                  cls_hbm, box_hbm, obj_hbm,
                  x_v, cls_v, box_v, obj_v, w_s, in_sem, out_sem):
    n_rows = x_hbm.shape[0]
    n_cls = cls_hbm.shape[0]
    n_box = box_hbm.shape[0]
    n_obj = obj_hbm.shape[0]
    chunks = _chunks(n_rows)

    def in_copy(c):
        s, ln = chunks[c]
        return pltpu.make_async_copy(
            x_hbm.at[pl.ds(s, ln), :], x_v.at[pl.ds(s, ln), :], in_sem.at[c])

    def out_copies(c):
        s, ln = chunks[c]
        sl = pl.ds(s, ln)
        return (
            pltpu.make_async_copy(cls_v.at[:, sl], cls_hbm.at[:, sl], out_sem.at[c, 0]),
            pltpu.make_async_copy(box_v.at[:, sl], box_hbm.at[:, sl], out_sem.at[c, 1]),
            pltpu.make_async_copy(obj_v.at[:, sl], obj_hbm.at[:, sl], out_sem.at[c, 2]),
        )

    for c in range(len(chunks)):
        in_copy(c).start()

    w_s[0:n_cls, :] = wc_ref[...]
    w_s[_OFF_BOX:_OFF_BOX + n_box, :] = wb_ref[...]
    w_s[_OFF_OBJ:_OFF_OBJ + n_obj, :] = wo_ref[...]
    b_col = jnp.transpose(b_ref[...])

    for c in range(len(chunks)):
        in_copy(c).wait()
        s, ln = chunks[c]
        acc = jax.lax.dot_general(
            w_s[...], x_v[pl.ds(s, ln), :], (((1,), (1,)), ((), ())),
            preferred_element_type=jnp.float32)
        acc = acc + b_col
        sl = pl.ds(s, ln)
        cls_v[:, sl] = acc[0:n_cls, :]
        box_v[:, sl] = acc[_OFF_BOX:_OFF_BOX + n_box, :]
        obj_v[:, sl] = acc[_OFF_OBJ:_OFF_OBJ + n_obj, :]
        for cp in out_copies(c):
            cp.start()

    for c in range(len(chunks)):
        for cp in out_copies(c):
            cp.wait()


def kernel(x, W_cls, b_cls, W_box, b_box, W_obj, b_obj):
    M, K = x.shape
    n_cls = W_cls.shape[1]
    n_box = W_box.shape[1]
    n_obj = W_obj.shape[1]

    WcT, WbT, WoT = W_cls.T, W_box.T, W_obj.T
    zc = jnp.zeros((_OFF_BOX - n_cls,), dtype=x.dtype)
    zb = jnp.zeros((_OFF_OBJ - _OFF_BOX - n_box,), dtype=x.dtype)
    zo = jnp.zeros((_NPAD - _OFF_OBJ - n_obj,), dtype=x.dtype)
    b_pad = jnp.concatenate([b_cls, zc, b_box, zb, b_obj, zo])[None, :]

    n_chunks = len(_chunks(M))
    cls_t, box_t, obj_t = pl.pallas_call(
        _heads_kernel,
        in_specs=[
            pl.BlockSpec(memory_space=pltpu.HBM),
            pl.BlockSpec(memory_space=pltpu.VMEM),
            pl.BlockSpec(memory_space=pltpu.VMEM),
            pl.BlockSpec(memory_space=pltpu.VMEM),
            pl.BlockSpec(memory_space=pltpu.VMEM),
        ],
        out_specs=[
            pl.BlockSpec(memory_space=pltpu.HBM),
            pl.BlockSpec(memory_space=pltpu.HBM),
            pl.BlockSpec(memory_space=pltpu.HBM),
        ],
        out_shape=[
            jax.ShapeDtypeStruct((n_cls, M), x.dtype),
            jax.ShapeDtypeStruct((n_box, M), x.dtype),
            jax.ShapeDtypeStruct((n_obj, M), x.dtype),
        ],
        scratch_shapes=[
            pltpu.VMEM((M, K), jnp.float32),
            pltpu.VMEM((n_cls, M), jnp.float32),
            pltpu.VMEM((n_box, M), jnp.float32),
            pltpu.VMEM((n_obj, M), jnp.float32),
            pltpu.VMEM((_NPAD, K), jnp.float32),
            pltpu.SemaphoreType.DMA((n_chunks,)),
            pltpu.SemaphoreType.DMA((n_chunks, 3)),
        ],
        compiler_params=pltpu.CompilerParams(
            vmem_limit_bytes=67108864,
        ),
    )(x, WcT, WbT, WoT, b_pad)
    return (cls_t.T, box_t.T, obj_t.T)


# speedup vs baseline: 1.0670x; 1.0670x over previous
import jax
import jax.numpy as jnp
from jax.experimental import pallas as pl
from jax.experimental.pallas import tpu as pltpu

_BM = 10112
_OFF_BOX = 24
_OFF_OBJ = 72
_NPAD = 80


def _heads_kernel(x_ref, wc_ref, wb_ref, wo_ref, b_ref,
                  cls_ref, box_ref, obj_ref, w_s):
    n_cls = cls_ref.shape[0]
    n_box = box_ref.shape[0]
    n_obj = obj_ref.shape[0]

    @pl.when(pl.program_id(0) == 0)
    def _init():
        w_s[...] = jnp.zeros_like(w_s)
        w_s[0:n_cls, :] = wc_ref[...]
        w_s[_OFF_BOX:_OFF_BOX + n_box, :] = wb_ref[...]
        w_s[_OFF_OBJ:_OFF_OBJ + n_obj, :] = wo_ref[...]

    acc = jax.lax.dot_general(
        w_s[...].astype(jnp.bfloat16), x_ref[...].astype(jnp.bfloat16),
        (((1,), (1,)), ((), ())),
        preferred_element_type=jnp.float32)
    acc = acc + jnp.transpose(b_ref[...])
    cls_ref[...] = acc[0:n_cls, :]
    box_ref[...] = acc[_OFF_BOX:_OFF_BOX + n_box, :]
    obj_ref[...] = acc[_OFF_OBJ:_OFF_OBJ + n_obj, :]


def kernel(x, W_cls, b_cls, W_box, b_box, W_obj, b_obj):
    M, K = x.shape
    n_cls = W_cls.shape[1]
    n_box = W_box.shape[1]
    n_obj = W_obj.shape[1]

    WcT, WbT, WoT = W_cls.T, W_box.T, W_obj.T
    zc = jnp.zeros((_OFF_BOX - n_cls,), dtype=x.dtype)
    zb = jnp.zeros((_OFF_OBJ - _OFF_BOX - n_box,), dtype=x.dtype)
    zo = jnp.zeros((_NPAD - _OFF_OBJ - n_obj,), dtype=x.dtype)
    b_pad = jnp.concatenate([b_cls, zc, b_box, zb, b_obj, zo])[None, :]

    grid = (pl.cdiv(M, _BM),)
    cls_t, box_t, obj_t = pl.pallas_call(
        _heads_kernel,
        grid=grid,
        in_specs=[
            pl.BlockSpec((_BM, K), lambda i: (i, 0)),
            pl.BlockSpec((n_cls, K), lambda i: (0, 0)),
            pl.BlockSpec((n_box, K), lambda i: (0, 0)),
            pl.BlockSpec((n_obj, K), lambda i: (0, 0)),
            pl.BlockSpec((1, _NPAD), lambda i: (0, 0)),
        ],
        out_specs=[
            pl.BlockSpec((n_cls, _BM), lambda i: (0, i)),
            pl.BlockSpec((n_box, _BM), lambda i: (0, i)),
            pl.BlockSpec((n_obj, _BM), lambda i: (0, i)),
        ],
        out_shape=[
            jax.ShapeDtypeStruct((n_cls, M), x.dtype),
            jax.ShapeDtypeStruct((n_box, M), x.dtype),
            jax.ShapeDtypeStruct((n_obj, M), x.dtype),
        ],
        scratch_shapes=[
            pltpu.VMEM((_NPAD, K), jnp.float32),
        ],
        compiler_params=pltpu.CompilerParams(
            dimension_semantics=("arbitrary",),
        ),
    )(x, WcT, WbT, WoT, b_pad)
    return (cls_t.T, box_t.T, obj_t.T)


# speedup vs baseline: 1.1975x; 1.1223x over previous
import jax
import jax.numpy as jnp
from jax.experimental import pallas as pl
from jax.experimental.pallas import tpu as pltpu

_BM = 10112
_OFF_BOX = 24
_OFF_OBJ = 72
_NPAD = 80


def _heads_kernel(x_ref, wc_ref, wb_ref, wo_ref, bc_ref, bb_ref, bo_ref,
                  cls_ref, box_ref, obj_ref, w_s, b_s):
    n_cls = cls_ref.shape[0]
    n_box = box_ref.shape[0]
    n_obj = obj_ref.shape[0]

    @pl.when(pl.program_id(0) == 0)
    def _init():
        w_s[...] = jnp.zeros_like(w_s)
        w_s[0:n_cls, :] = wc_ref[...]
        w_s[_OFF_BOX:_OFF_BOX + n_box, :] = wb_ref[...]
        w_s[_OFF_OBJ:_OFF_OBJ + n_obj, :] = wo_ref[...]
        b_s[...] = jnp.zeros_like(b_s)
        b_s[0:1, 0:n_cls] = bc_ref[...][None, :]
        b_s[0:1, _OFF_BOX:_OFF_BOX + n_box] = bb_ref[...][None, :]
        b_s[0:1, _OFF_OBJ:_OFF_OBJ + n_obj] = bo_ref[...][None, :]

    acc = jax.lax.dot_general(
        w_s[...], x_ref[...], (((1,), (1,)), ((), ())),
        preferred_element_type=jnp.float32)
    acc = acc + jnp.transpose(b_s[...])
    cls_ref[...] = acc[0:n_cls, :]
    box_ref[...] = acc[_OFF_BOX:_OFF_BOX + n_box, :]
    obj_ref[...] = acc[_OFF_OBJ:_OFF_OBJ + n_obj, :]


def kernel(x, W_cls, b_cls, W_box, b_box, W_obj, b_obj):
    M, K = x.shape
    n_cls = W_cls.shape[1]
    n_box = W_box.shape[1]
    n_obj = W_obj.shape[1]

    WcT, WbT, WoT = W_cls.T, W_box.T, W_obj.T

    grid = (pl.cdiv(M, _BM),)
    cls_t, box_t, obj_t = pl.pallas_call(
        _heads_kernel,
        grid=grid,
        in_specs=[
            pl.BlockSpec((_BM, K), lambda i: (i, 0)),
            pl.BlockSpec((n_cls, K), lambda i: (0, 0)),
            pl.BlockSpec((n_box, K), lambda i: (0, 0)),
            pl.BlockSpec((n_obj, K), lambda i: (0, 0)),
            pl.BlockSpec(memory_space=pltpu.VMEM),
            pl.BlockSpec(memory_space=pltpu.VMEM),
            pl.BlockSpec(memory_space=pltpu.VMEM),
        ],
        out_specs=[
            pl.BlockSpec((n_cls, _BM), lambda i: (0, i)),
            pl.BlockSpec((n_box, _BM), lambda i: (0, i)),
            pl.BlockSpec((n_obj, _BM), lambda i: (0, i)),
        ],
        out_shape=[
            jax.ShapeDtypeStruct((n_cls, M), x.dtype),
            jax.ShapeDtypeStruct((n_box, M), x.dtype),
            jax.ShapeDtypeStruct((n_obj, M), x.dtype),
        ],
        scratch_shapes=[
            pltpu.VMEM((_NPAD, K), jnp.float32),
            pltpu.VMEM((1, _NPAD), jnp.float32),
        ],
        compiler_params=pltpu.CompilerParams(
            dimension_semantics=("arbitrary",),
        ),
    )(x, WcT, WbT, WoT, b_cls, b_box, b_obj)
    return (cls_t.T, box_t.T, obj_t.T)
